# initial kernel scaffold (unmeasured)
import jax
import jax.numpy as jnp
from jax import lax
from jax.experimental import pallas as pl
from jax.experimental.pallas import tpu as pltpu


def kernel(
    x,
):
    def body(*refs):
        pass

    out_shape = jax.ShapeDtypeStruct(..., jnp.float32)
    return pl.pallas_call(body, out_shape=out_shape)(...)


# baseline (device time: 22954 ns/iter reference)
import jax
import jax.numpy as jnp
from jax import lax
from jax.experimental import pallas as pl
from jax.experimental.pallas import tpu as pltpu

N_DEV = 16


def kernel(x):
    m, n = x.shape

    B = 256
    NB = m // B

    def body(x_ref, out_ref, own_ref, totals_ref, send_sems, recv_sems):
        my = lax.axis_index("i")

        row = lax.broadcasted_iota(jnp.int32, (B, B), 0)
        col = lax.broadcasted_iota(jnp.int32, (B, B), 1)
        ltri = (row >= col).astype(jnp.bfloat16)
        x_bf = x_ref[:, :].astype(jnp.bfloat16)
        carry = jnp.zeros((1, n), jnp.float32)
        for b in range(NB):
            cs = lax.dot_general(
                ltri,
                x_bf[b * B : (b + 1) * B, :],
                (((1,), (0,)), ((), ())),
                preferred_element_type=jnp.float32,
            )
            cs = cs + carry
            out_ref[pl.ds(b * B, B), :] = cs
            carry = cs[B - 1 : B, :]
        own_ref[:, :] = carry

        for p in range(N_DEV):
            @pl.when(my != p)
            def _(p=p):
                rdma = pltpu.make_async_remote_copy(
                    src_ref=own_ref,
                    dst_ref=totals_ref.at[pl.ds(my, 1)],
                    send_sem=send_sems.at[p],
                    recv_sem=recv_sems.at[my],
                    device_id=(p,),
                    device_id_type=pl.DeviceIdType.MESH,
                )
                rdma.start()

        for p in range(N_DEV):
            @pl.when(my != p)
            def _(p=p):
                recv = pltpu.make_async_remote_copy(
                    src_ref=own_ref,
                    dst_ref=totals_ref.at[pl.ds(p, 1)],
                    send_sem=send_sems.at[p],
                    recv_sem=recv_sems.at[p],
                    device_id=(p,),
                    device_id_type=pl.DeviceIdType.MESH,
                )
                recv.wait_recv()
                send = pltpu.make_async_remote_copy(
                    src_ref=own_ref,
                    dst_ref=totals_ref.at[pl.ds(my, 1)],
                    send_sem=send_sems.at[p],
                    recv_sem=recv_sems.at[my],
                    device_id=(p,),
                    device_id_type=pl.DeviceIdType.MESH,
                )
                send.wait_send()

        pos = lax.broadcasted_iota(jnp.int32, (N_DEV, n), 0)
        offset = jnp.sum(
            jnp.where(pos < my, totals_ref[:, :], 0.0), axis=0
        )
        out_ref[:, :] = out_ref[:, :] + offset[None, :]

    return pl.pallas_call(
        body,
        out_shape=jax.ShapeDtypeStruct((m, n), x.dtype),
        in_specs=[pl.BlockSpec(memory_space=pltpu.VMEM)],
        out_specs=pl.BlockSpec(memory_space=pltpu.VMEM),
        scratch_shapes=[
            pltpu.VMEM((1, n), x.dtype),
            pltpu.VMEM((N_DEV, n), x.dtype),
            pltpu.SemaphoreType.DMA((N_DEV,)),
            pltpu.SemaphoreType.DMA((N_DEV,)),
        ],
    )(x)


# device time: 22303 ns/iter; 1.0292x vs baseline; 1.0292x over previous
import jax
import jax.numpy as jnp
from jax import lax
from jax.experimental import pallas as pl
from jax.experimental.pallas import tpu as pltpu

N_DEV = 16


def kernel(x):
    m, n = x.shape

    B = 256
    NB = m // B

    def body(x_ref, out_ref, own_ref, totals_ref, send_sems, recv_sems):
        my = lax.axis_index("i")

        x_bf = x_ref[:, :].astype(jnp.bfloat16)
        own_ref[:, :] = jnp.sum(
            x_bf.astype(jnp.float32), axis=0, keepdims=True
        )

        for p in range(N_DEV):
            @pl.when(my != p)
            def _(p=p):
                rdma = pltpu.make_async_remote_copy(
                    src_ref=own_ref,
                    dst_ref=totals_ref.at[pl.ds(my, 1)],
                    send_sem=send_sems.at[p],
                    recv_sem=recv_sems.at[my],
                    device_id=(p,),
                    device_id_type=pl.DeviceIdType.MESH,
                )
                rdma.start()

        row = lax.broadcasted_iota(jnp.int32, (B, B), 0)
        col = lax.broadcasted_iota(jnp.int32, (B, B), 1)
        ltri = (row >= col).astype(jnp.bfloat16)
        carry = jnp.zeros((1, n), jnp.float32)
        for b in range(NB):
            cs = lax.dot_general(
                ltri,
                x_bf[b * B : (b + 1) * B, :],
                (((1,), (0,)), ((), ())),
                preferred_element_type=jnp.float32,
            )
            cs = cs + carry
            out_ref[pl.ds(b * B, B), :] = cs
            carry = cs[B - 1 : B, :]

        for p in range(N_DEV):
            @pl.when(my != p)
            def _(p=p):
                recv = pltpu.make_async_remote_copy(
                    src_ref=own_ref,
                    dst_ref=totals_ref.at[pl.ds(p, 1)],
                    send_sem=send_sems.at[p],
                    recv_sem=recv_sems.at[p],
                    device_id=(p,),
                    device_id_type=pl.DeviceIdType.MESH,
                )
                recv.wait_recv()
                send = pltpu.make_async_remote_copy(
                    src_ref=own_ref,
                    dst_ref=totals_ref.at[pl.ds(my, 1)],
                    send_sem=send_sems.at[p],
                    recv_sem=recv_sems.at[my],
                    device_id=(p,),
                    device_id_type=pl.DeviceIdType.MESH,
                )
                send.wait_send()

        pos = lax.broadcasted_iota(jnp.int32, (N_DEV, n), 0)
        offset = jnp.sum(
            jnp.where(pos < my, totals_ref[:, :], 0.0), axis=0
        )
        out_ref[:, :] = out_ref[:, :] + offset[None, :]

    return pl.pallas_call(
        body,
        out_shape=jax.ShapeDtypeStruct((m, n), x.dtype),
        in_specs=[pl.BlockSpec(memory_space=pltpu.VMEM)],
        out_specs=pl.BlockSpec(memory_space=pltpu.VMEM),
        scratch_shapes=[
            pltpu.VMEM((1, n), x.dtype),
            pltpu.VMEM((N_DEV, n), x.dtype),
            pltpu.SemaphoreType.DMA((N_DEV,)),
            pltpu.SemaphoreType.DMA((N_DEV,)),
        ],
    )(x)


# device time: 15090 ns/iter; 1.5211x vs baseline; 1.4780x over previous
import jax
import jax.numpy as jnp
from jax import lax
from jax.experimental import pallas as pl
from jax.experimental.pallas import tpu as pltpu

N_DEV = 16
N_ROUNDS = 4


def build_kernel(block=256, comm="hillis", do_cumsum=True, do_pass2=True):
    def kernel(x):
        m, n = x.shape
        B = block
        NB = m // B

        def body(x_ref, out_ref, own_ref, cur_ref, send_buf, recv_buf,
                 totals_ref, send_sems, recv_sems):
            my = lax.axis_index("i")

            x_bf = x_ref[:, :].astype(jnp.bfloat16)
            own_ref[:, :] = jnp.sum(
                x_bf.astype(jnp.float32), axis=0, keepdims=True
            )

            if comm == "a2ax":
                barrier_sem = pltpu.get_barrier_semaphore()
                for p in range(N_DEV):
                    @pl.when(my != p)
                    def _(p=p):
                        pl.semaphore_signal(
                            barrier_sem, inc=1, device_id=(p,),
                            device_id_type=pl.DeviceIdType.MESH,
                        )
                pl.semaphore_wait(barrier_sem, N_DEV - 1)

            if comm in ("hillisx", "hillisv", "hillisf", "ring1x"):
                barrier_sem = pltpu.get_barrier_semaphore()
                if comm in ("hillisx", "hillisv", "hillisf"):
                    for r in range(N_ROUNDS):
                        d = 1 << r

                        @pl.when(my - d >= 0)
                        def _(d=d):
                            pl.semaphore_signal(
                                barrier_sem, inc=1,
                                device_id=(my - d,),
                                device_id_type=pl.DeviceIdType.MESH,
                            )
                    if comm in ("hillisx", "hillisf"):
                        for r in range(N_ROUNDS):
                            d = 1 << r

                            @pl.when(my + d < N_DEV)
                            def _():
                                pl.semaphore_wait(barrier_sem, 1)
                else:
                    left = jnp.where(my == 0, N_DEV - 1, my - 1)
                    right = jnp.where(my == N_DEV - 1, 0, my + 1)
                    pl.semaphore_signal(
                        barrier_sem, inc=1, device_id=(left,),
                        device_id_type=pl.DeviceIdType.MESH,
                    )
                    pl.semaphore_wait(barrier_sem, 1)

            if comm in ("ring1", "ring1x"):
                right = jnp.where(my == N_DEV - 1, 0, my + 1)
                rdma = pltpu.make_async_remote_copy(
                    src_ref=own_ref,
                    dst_ref=totals_ref.at[pl.ds(0, 1)],
                    send_sem=send_sems.at[0],
                    recv_sem=recv_sems.at[0],
                    device_id=(right,),
                    device_id_type=pl.DeviceIdType.MESH,
                )
                rdma.start()
                rdma.wait_recv()
                rdma.wait_send()

            if comm in ("a2a", "a2ax"):
                for p in range(N_DEV):
                    @pl.when(my != p)
                    def _(p=p):
                        rdma = pltpu.make_async_remote_copy(
                            src_ref=own_ref,
                            dst_ref=totals_ref.at[pl.ds(my, 1)],
                            send_sem=send_sems.at[p],
                            recv_sem=recv_sems.at[my],
                            device_id=(p,),
                            device_id_type=pl.DeviceIdType.MESH,
                        )
                        rdma.start()
            elif comm in ("hillis", "hillisx", "hillisv", "hillisf"):
                cur_ref[:, :] = own_ref[:, :]

            def hillis_send(r):
                d = 1 << r
                send_buf[r, :] = cur_ref[0, :]

                @pl.when(my + d < N_DEV)
                def _():
                    rdma = pltpu.make_async_remote_copy(
                        src_ref=send_buf.at[pl.ds(r, 1)],
                        dst_ref=recv_buf.at[pl.ds(r, 1)],
                        send_sem=send_sems.at[r],
                        recv_sem=recv_sems.at[r],
                        device_id=(my + d,),
                        device_id_type=pl.DeviceIdType.MESH,
                    )
                    rdma.start()

            def hillis_recv(r):
                d = 1 << r

                @pl.when(my >= d)
                def _():
                    recv = pltpu.make_async_remote_copy(
                        src_ref=send_buf.at[pl.ds(r, 1)],
                        dst_ref=recv_buf.at[pl.ds(r, 1)],
                        send_sem=send_sems.at[r],
                        recv_sem=recv_sems.at[r],
                        device_id=(my - d,),
                        device_id_type=pl.DeviceIdType.MESH,
                    )
                    recv.wait_recv()
                    cur_ref[:, :] = cur_ref[:, :] + recv_buf[r : r + 1, :]

            if comm in ("hillis", "hillisx", "hillisf"):
                for r in range(N_ROUNDS):
                    hillis_send(r)
                    hillis_recv(r)

            if comm == "hillisf":
                row = lax.broadcasted_iota(jnp.int32, (B, B), 0)
                col = lax.broadcasted_iota(jnp.int32, (B, B), 1)
                ltri = (row >= col).astype(jnp.bfloat16)
                carry = cur_ref[:, :] - own_ref[:, :]
                for b in range(NB):
                    cs = lax.dot_general(
                        ltri,
                        x_bf[b * B : (b + 1) * B, :],
                        (((1,), (0,)), ((), ())),
                        preferred_element_type=jnp.float32,
                    )
                    cs = cs + carry
                    out_ref[pl.ds(b * B, B), :] = cs
                    carry = cs[B - 1 : B, :]

            elif comm == "hillisv":
                row = lax.broadcasted_iota(jnp.int32, (B, B), 0)
                col = lax.broadcasted_iota(jnp.int32, (B, B), 1)
                ltri = (row >= col).astype(jnp.bfloat16)
                carry = jnp.zeros((1, n), jnp.float32)

                def mm_block(b, carry):
                    cs = lax.dot_general(
                        ltri,
                        x_bf[b * B : (b + 1) * B, :],
                        (((1,), (0,)), ((), ())),
                        preferred_element_type=jnp.float32,
                    )
                    cs = cs + carry
                    out_ref[pl.ds(b * B, B), :] = cs
                    return cs[B - 1 : B, :]

                schedule = {0: [1], 1: [2, 3], 2: [4, 5], 3: [6, 7]}
                carry = mm_block(0, carry)
                for r in range(N_ROUNDS):
                    d = 1 << r

                    @pl.when(my + d < N_DEV)
                    def _():
                        pl.semaphore_wait(barrier_sem, 1)
                for r in range(N_ROUNDS):
                    hillis_send(r)
                    for b in schedule[r]:
                        carry = mm_block(b, carry)
                    hillis_recv(r)

            elif do_cumsum:
                row = lax.broadcasted_iota(jnp.int32, (B, B), 0)
                col = lax.broadcasted_iota(jnp.int32, (B, B), 1)
                ltri = (row >= col).astype(jnp.bfloat16)
                carry = jnp.zeros((1, n), jnp.float32)
                for b in range(NB):
                    cs = lax.dot_general(
                        ltri,
                        x_bf[b * B : (b + 1) * B, :],
                        (((1,), (0,)), ((), ())),
                        preferred_element_type=jnp.float32,
                    )
                    cs = cs + carry
                    out_ref[pl.ds(b * B, B), :] = cs
                    carry = cs[B - 1 : B, :]
            else:
                out_ref[:, :] = x_ref[:, :]

            if comm in ("a2a", "a2ax"):
                for p in range(N_DEV):
                    @pl.when(my != p)
                    def _(p=p):
                        recv = pltpu.make_async_remote_copy(
                            src_ref=own_ref,
                            dst_ref=totals_ref.at[pl.ds(p, 1)],
                            send_sem=send_sems.at[p],
                            recv_sem=recv_sems.at[p],
                            device_id=(p,),
                            device_id_type=pl.DeviceIdType.MESH,
                        )
                        recv.wait_recv()
                        send = pltpu.make_async_remote_copy(
                            src_ref=own_ref,
                            dst_ref=totals_ref.at[pl.ds(my, 1)],
                            send_sem=send_sems.at[p],
                            recv_sem=recv_sems.at[my],
                            device_id=(p,),
                            device_id_type=pl.DeviceIdType.MESH,
                        )
                        send.wait_send()
            elif comm in ("hillis", "hillisx", "hillisv", "hillisf"):
                for r in range(N_ROUNDS):
                    d = 1 << r

                    @pl.when(my + d < N_DEV)
                    def _(r=r, d=d):
                        send = pltpu.make_async_remote_copy(
                            src_ref=send_buf.at[pl.ds(r, 1)],
                            dst_ref=recv_buf.at[pl.ds(r, 1)],
                            send_sem=send_sems.at[r],
                            recv_sem=recv_sems.at[r],
                            device_id=(my + d,),
                            device_id_type=pl.DeviceIdType.MESH,
                        )
                        send.wait_send()

            if do_pass2 and comm != "hillisf":
                if comm in ("a2a", "a2ax"):
                    pos = lax.broadcasted_iota(jnp.int32, (N_DEV, n), 0)
                    offset = jnp.sum(
                        jnp.where(pos < my, totals_ref[:, :], 0.0), axis=0
                    )[None, :]
                elif comm in ("hillis", "hillisx", "hillisv"):
                    offset = cur_ref[:, :] - own_ref[:, :]
                else:
                    offset = jnp.zeros((1, n), jnp.float32)
                out_ref[:, :] = out_ref[:, :] + offset

        params = (
            dict(compiler_params=pltpu.CompilerParams(collective_id=0))
            if comm in ("hillisx", "hillisv", "hillisf", "ring1x", "a2ax")
            else {}
        )
        return pl.pallas_call(
            body,
            **params,
            out_shape=jax.ShapeDtypeStruct((m, n), x.dtype),
            in_specs=[pl.BlockSpec(memory_space=pltpu.VMEM)],
            out_specs=pl.BlockSpec(memory_space=pltpu.VMEM),
            scratch_shapes=[
                pltpu.VMEM((1, n), x.dtype),
                pltpu.VMEM((1, n), x.dtype),
                pltpu.VMEM((N_ROUNDS, n), x.dtype),
                pltpu.VMEM((N_ROUNDS, n), x.dtype),
                pltpu.VMEM((N_DEV, n), x.dtype),
                pltpu.SemaphoreType.DMA((N_DEV,)),
                pltpu.SemaphoreType.DMA((N_DEV,)),
            ],
        )(x)

    return kernel


kernel = build_kernel(comm="hillisf")


# device time: 14914 ns/iter; 1.5391x vs baseline; 1.0118x over previous
import jax
import jax.numpy as jnp
from jax import lax
from jax.experimental import pallas as pl
from jax.experimental.pallas import tpu as pltpu

N_DEV = 16
N_ROUNDS = 4
BLOCK = 256


def kernel(x):
    m, n = x.shape
    B = BLOCK
    NB = m // B

    def body(x_ref, out_ref, own_ref, cur_ref, send_buf, recv_buf,
             send_sems, recv_sems):
        my = lax.axis_index("i")

        x_bf = x_ref[:, :].astype(jnp.bfloat16)
        own_ref[:, :] = jnp.sum(
            x_bf.astype(jnp.float32), axis=0, keepdims=True
        )
        cur_ref[:, :] = own_ref[:, :]

        barrier_sem = pltpu.get_barrier_semaphore()
        for r in range(N_ROUNDS):
            d = 1 << r

            @pl.when(my - d >= 0)
            def _(d=d):
                pl.semaphore_signal(
                    barrier_sem, inc=1,
                    device_id=(my - d,),
                    device_id_type=pl.DeviceIdType.MESH,
                )
        for r in range(N_ROUNDS):
            d = 1 << r

            @pl.when(my + d < N_DEV)
            def _():
                pl.semaphore_wait(barrier_sem, 1)

        for r in range(N_ROUNDS):
            d = 1 << r
            send_buf[r, :] = cur_ref[0, :]

            @pl.when(my + d < N_DEV)
            def _(r=r, d=d):
                rdma = pltpu.make_async_remote_copy(
                    src_ref=send_buf.at[pl.ds(r, 1)],
                    dst_ref=recv_buf.at[pl.ds(r, 1)],
                    send_sem=send_sems.at[r],
                    recv_sem=recv_sems.at[r],
                    device_id=(my + d,),
                    device_id_type=pl.DeviceIdType.MESH,
                )
                rdma.start()

            @pl.when(my >= d)
            def _(r=r, d=d):
                recv = pltpu.make_async_remote_copy(
                    src_ref=send_buf.at[pl.ds(r, 1)],
                    dst_ref=recv_buf.at[pl.ds(r, 1)],
                    send_sem=send_sems.at[r],
                    recv_sem=recv_sems.at[r],
                    device_id=(my - d,),
                    device_id_type=pl.DeviceIdType.MESH,
                )
                recv.wait_recv()
                cur_ref[:, :] = cur_ref[:, :] + recv_buf[r : r + 1, :]

        row = lax.broadcasted_iota(jnp.int32, (B, B), 0)
        col = lax.broadcasted_iota(jnp.int32, (B, B), 1)
        ltri = (row >= col).astype(jnp.bfloat16)
        carry = cur_ref[:, :] - own_ref[:, :]
        for b in range(NB):
            cs = lax.dot_general(
                ltri,
                x_bf[b * B : (b + 1) * B, :],
                (((1,), (0,)), ((), ())),
                preferred_element_type=jnp.float32,
            )
            cs = cs + carry
            out_ref[pl.ds(b * B, B), :] = cs
            carry = cs[B - 1 : B, :]

        for r in range(N_ROUNDS):
            d = 1 << r

            @pl.when(my + d < N_DEV)
            def _(r=r, d=d):
                send = pltpu.make_async_remote_copy(
                    src_ref=send_buf.at[pl.ds(r, 1)],
                    dst_ref=recv_buf.at[pl.ds(r, 1)],
                    send_sem=send_sems.at[r],
                    recv_sem=recv_sems.at[r],
                    device_id=(my + d,),
                    device_id_type=pl.DeviceIdType.MESH,
                )
                send.wait_send()

    return pl.pallas_call(
        body,
        out_shape=jax.ShapeDtypeStruct((m, n), x.dtype),
        in_specs=[pl.BlockSpec(memory_space=pltpu.VMEM)],
        out_specs=pl.BlockSpec(memory_space=pltpu.VMEM),
        scratch_shapes=[
            pltpu.VMEM((1, n), x.dtype),
            pltpu.VMEM((1, n), x.dtype),
            pltpu.VMEM((N_ROUNDS, n), x.dtype),
            pltpu.VMEM((N_ROUNDS, n), x.dtype),
            pltpu.SemaphoreType.DMA((N_ROUNDS,)),
            pltpu.SemaphoreType.DMA((N_ROUNDS,)),
        ],
        compiler_params=pltpu.CompilerParams(collective_id=0),
    )(x)
